# baseline (speedup 1.0000x reference)
import functools

import jax
import jax.numpy as jnp
from jax.experimental import pallas as pl
from jax.experimental.pallas import tpu as pltpu

_INTERPRET = False


def _leaky(x):
    return jnp.where(x >= 0, x, 0.01 * x)


def _bdot(a, b):
    return jnp.dot(a.astype(jnp.bfloat16), b.astype(jnp.bfloat16),
                   preferred_element_type=jnp.float32)


def _mm_body(x_ref, w_ref, o_ref):
    o_ref[...] = _bdot(x_ref[...], w_ref[...])


def _matmul(x, W, bm):
    M, K = x.shape
    N = W.shape[1]
    return pl.pallas_call(
        _mm_body,
        grid=(pl.cdiv(M, bm),),
        in_specs=[pl.BlockSpec((bm, K), lambda i: (i, 0)),
                  pl.BlockSpec((K, N), lambda i: (0, 0))],
        out_specs=pl.BlockSpec((bm, N), lambda i: (i, 0)),
        out_shape=jax.ShapeDtypeStruct((M, N), jnp.float32),
        interpret=_INTERPRET,
    )(x, W)


def _mlp_body(dv_ref, pe_ref, ec_ref, go_ref,
              We1_ref, be1_ref, We2_ref, be2_ref,
              Wd1_ref, bd1_ref, Wd2_ref, bd2_ref,
              Wo1_ref, bo1_ref,
              feat_ref, enc_ref, dec_ref, h_ref, stats_ref,
              acc_ref):
    feat = jnp.concatenate(
        [dv_ref[...], pe_ref[...], ec_ref[...], go_ref[...]], axis=1)
    feat_ref[...] = feat
    e1 = _leaky(_bdot(feat, We1_ref[...]) + be1_ref[...])
    enc = _leaky(_bdot(e1, We2_ref[...]) + be2_ref[...])
    enc_ref[...] = enc
    d1 = _leaky(_bdot(enc, Wd1_ref[...]) + bd1_ref[...])
    dec_ref[...] = _leaky(_bdot(d1, Wd2_ref[...]) + bd2_ref[...])
    h = jnp.dot(enc, Wo1_ref[...],
                preferred_element_type=jnp.float32) + bo1_ref[...]
    h_ref[...] = h

    i = pl.program_id(0)

    @pl.when(i == 0)
    def _init():
        acc_ref[...] = jnp.zeros_like(acc_ref)

    acc_ref[0, :] += jnp.sum(h, axis=0)
    acc_ref[1, :] += jnp.sum(h * h, axis=0)

    @pl.when(i == pl.num_programs(0) - 1)
    def _emit():
        stats_ref[...] = acc_ref[...]


def _mlp(d_vecs, p_embeddings, ecfps_g, gos_g,
         We1, be1, We2, be2, Wdec1, bdec1, Wdec2, bdec2, Wo1, bo1, bm):
    Bn = d_vecs.shape[0]
    F0 = d_vecs.shape[1]
    F1 = p_embeddings.shape[1]
    F2 = ecfps_g.shape[1]
    F3 = gos_g.shape[1]
    FEAT = F0 + F1 + F2 + F3
    H1 = We1.shape[1]
    H2 = We2.shape[1]
    D1 = Wdec1.shape[1]
    D2 = Wdec2.shape[1]
    HO = Wo1.shape[1]
    grid = (Bn // bm,)

    def row_block(i):
        return (i, 0)

    def const_block(i):
        return (0, 0)

    def vec_block(i):
        return (0,)

    out_shapes = (
        jax.ShapeDtypeStruct((Bn, FEAT), jnp.float32),
        jax.ShapeDtypeStruct((Bn, H2), jnp.float32),
        jax.ShapeDtypeStruct((Bn, D2), jnp.float32),
        jax.ShapeDtypeStruct((Bn, HO), jnp.float32),
        jax.ShapeDtypeStruct((2, HO), jnp.float32),
    )
    out_specs = (
        pl.BlockSpec((bm, FEAT), row_block),
        pl.BlockSpec((bm, H2), row_block),
        pl.BlockSpec((bm, D2), row_block),
        pl.BlockSpec((bm, HO), row_block),
        pl.BlockSpec((2, HO), const_block),
    )
    in_specs = [
        pl.BlockSpec((bm, F0), row_block),
        pl.BlockSpec((bm, F1), row_block),
        pl.BlockSpec((bm, F2), row_block),
        pl.BlockSpec((bm, F3), row_block),
        pl.BlockSpec((FEAT, H1), const_block),
        pl.BlockSpec((H1,), vec_block),
        pl.BlockSpec((H1, H2), const_block),
        pl.BlockSpec((H2,), vec_block),
        pl.BlockSpec((H2, D1), const_block),
        pl.BlockSpec((D1,), vec_block),
        pl.BlockSpec((D1, D2), const_block),
        pl.BlockSpec((D2,), vec_block),
        pl.BlockSpec((H2, HO), const_block),
        pl.BlockSpec((HO,), vec_block),
    ]
    return pl.pallas_call(
        _mlp_body,
        grid=grid,
        in_specs=in_specs,
        out_specs=out_specs,
        out_shape=out_shapes,
        scratch_shapes=[pltpu.VMEM((2, HO), jnp.float32)],
        interpret=_INTERPRET,
    )(d_vecs, p_embeddings, ecfps_g, gos_g,
      We1, be1, We2, be2, Wdec1, bdec1, Wdec2, bdec2, Wo1, bo1)


def _head_body(h_ref, stats_ref, gamma_ref, beta_ref, Wo2_ref, bo2_ref,
               y_ref, *, inv_b):
    mean = stats_ref[0, :] * inv_b
    var = stats_ref[1, :] * inv_b - mean * mean
    hn = (h_ref[...] - mean) * jax.lax.rsqrt(var + 1e-5) * gamma_ref[...] \
        + beta_ref[...]
    hn = _leaky(hn)
    y_ref[...] = jnp.dot(hn, Wo2_ref[...],
                         preferred_element_type=jnp.float32) + bo2_ref[...]


def _head(h, stats, gamma, beta, Wo2, bo2, bm):
    Bn, HO = h.shape
    grid = (Bn // bm,)
    return pl.pallas_call(
        functools.partial(_head_body, inv_b=1.0 / Bn),
        grid=grid,
        in_specs=[
            pl.BlockSpec((bm, HO), lambda i: (i, 0)),
            pl.BlockSpec((2, HO), lambda i: (0, 0)),
            pl.BlockSpec((HO,), lambda i: (0,)),
            pl.BlockSpec((HO,), lambda i: (0,)),
            pl.BlockSpec((HO, 1), lambda i: (0, 0)),
            pl.BlockSpec((1,), lambda i: (0,)),
        ],
        out_specs=pl.BlockSpec((bm, 1), lambda i: (i, 0)),
        out_shape=jax.ShapeDtypeStruct((Bn, 1), jnp.float32),
        interpret=_INTERPRET,
    )(h, stats, gamma, beta, Wo2, bo2)


def _gcn_aggregate(h, edge_index, edge_weight, b):
    n = h.shape[0]
    src = edge_index[0]
    dst = edge_index[1]
    deg = jnp.zeros((n,), jnp.float32).at[dst].add(edge_weight) + 1.0
    dinv = jax.lax.rsqrt(deg)
    coeff = dinv[src] * edge_weight * dinv[dst]
    out = (h * (dinv * dinv)[:, None]).at[dst].add(coeff[:, None] * h[src])
    return _leaky(out + b)


def kernel(d_index, p_index, d_vecs, p_embeddings, y,
           d_ecfps, d_edge_index, d_edge_weight,
           p_gos, p_edge_index, p_edge_weight,
           Wd, bd, Wp, bp,
           We1, be1, We2, be2,
           Wdec1, bdec1, Wdec2, bdec2,
           Wo1, bo1, gamma, beta, Wo2, bo2):
    h_d = _matmul(d_ecfps, Wd, bm=400)
    h_p = _matmul(p_gos, Wp, bm=400)

    out_d = _gcn_aggregate(h_d, d_edge_index, d_edge_weight, bd)
    out_p = _gcn_aggregate(h_p, p_edge_index, p_edge_weight, bp)
    ecfps_g = out_d[d_index]
    gos_g = out_p[p_index]

    feature, encoded, decoded, h, stats = _mlp(
        d_vecs, p_embeddings, ecfps_g, gos_g,
        We1, be1, We2, be2, Wdec1, bdec1, Wdec2, bdec2, Wo1, bo1,
        bm=min(256, d_vecs.shape[0]))
    y_out = _head(h, stats, gamma, beta, Wo2, bo2,
                  bm=min(512, d_vecs.shape[0]))
    return (y_out, encoded, decoded, feature)


# speedup vs baseline: 1.1659x; 1.1659x over previous
import functools

import jax
import jax.numpy as jnp
from jax import lax
from jax.experimental import pallas as pl
from jax.experimental.pallas import tpu as pltpu
from jax.experimental.pallas import tpu_sc as plsc

_INTERPRET = False

_N = 10000
_F = 1024
_CH = 2000
_NSTEP = 3


def _leaky(x):
    return jnp.where(x >= 0, x, 0.01 * x)


def _bdot(a, b):
    return jnp.dot(a.astype(jnp.bfloat16), b.astype(jnp.bfloat16),
                   preferred_element_type=jnp.float32)


def _mm_body(x_ref, w_ref, dinv_ref, o_ref):
    f = pl.program_id(1)
    res = jnp.dot(x_ref[...], w_ref[:, pl.ds(f * 128, 128)],
                  preferred_element_type=jnp.float32)
    o_ref[...] = res * dinv_ref[...][:, None]


def _matmul_scaled(x, W, dinv):
    M, K = x.shape
    bm = 512
    nb = pl.cdiv(_NPAD, bm)
    return pl.pallas_call(
        _mm_body,
        grid=(nb, 8),
        in_specs=[pl.BlockSpec((bm, K), lambda i, f: (i, 0)),
                  pl.BlockSpec((K, _F), lambda i, f: (0, 0)),
                  pl.BlockSpec((bm,), lambda i, f: (i,))],
        out_specs=pl.BlockSpec((bm, 128), lambda i, f: (f * nb + i, 0)),
        out_shape=jax.ShapeDtypeStruct((8 * _NPAD, 128), jnp.float32),
        interpret=_INTERPRET,
    )(x.astype(jnp.bfloat16), W.astype(jnp.bfloat16), dinv)


def _mlp_body(dv_ref, pe_ref, ec_ref, go_ref,
              We1_ref, be1_ref, We2_ref, be2_ref,
              Wd1_ref, bd1_ref, Wd2_ref, bd2_ref,
              Wo1_ref, bo1_ref,
              feat_ref, enc_ref, dec_ref, h_ref, stats_ref,
              acc_ref):
    feat = jnp.concatenate(
        [dv_ref[...], pe_ref[...], ec_ref[...], go_ref[...]], axis=1)
    feat_ref[...] = feat
    e1 = _leaky(_bdot(feat, We1_ref[...]) + be1_ref[...])
    enc = _leaky(_bdot(e1, We2_ref[...]) + be2_ref[...])
    enc_ref[...] = enc
    d1 = _leaky(_bdot(enc, Wd1_ref[...]) + bd1_ref[...])
    dec_ref[...] = _leaky(_bdot(d1, Wd2_ref[...]) + bd2_ref[...])
    h = jnp.dot(enc, Wo1_ref[...],
                preferred_element_type=jnp.float32) + bo1_ref[...]
    h_ref[...] = h

    i = pl.program_id(0)

    @pl.when(i == 0)
    def _init():
        acc_ref[...] = jnp.zeros_like(acc_ref)

    acc_ref[0, :] += jnp.sum(h, axis=0)
    acc_ref[1, :] += jnp.sum(h * h, axis=0)

    @pl.when(i == pl.num_programs(0) - 1)
    def _emit():
        stats_ref[...] = acc_ref[...]


def _mlp(d_vecs, p_embeddings, ecfps_g, gos_g,
         We1, be1, We2, be2, Wdec1, bdec1, Wdec2, bdec2, Wo1, bo1, bm):
    Bn = d_vecs.shape[0]
    F0 = d_vecs.shape[1]
    F1 = p_embeddings.shape[1]
    F2 = ecfps_g.shape[1]
    F3 = gos_g.shape[1]
    FEAT = F0 + F1 + F2 + F3
    H1 = We1.shape[1]
    H2 = We2.shape[1]
    D1 = Wdec1.shape[1]
    D2 = Wdec2.shape[1]
    HO = Wo1.shape[1]
    grid = (Bn // bm,)

    def row_block(i):
        return (i, 0)

    def const_block(i):
        return (0, 0)

    def vec_block(i):
        return (0,)

    out_shapes = (
        jax.ShapeDtypeStruct((Bn, FEAT), jnp.float32),
        jax.ShapeDtypeStruct((Bn, H2), jnp.float32),
        jax.ShapeDtypeStruct((Bn, D2), jnp.float32),
        jax.ShapeDtypeStruct((Bn, HO), jnp.float32),
        jax.ShapeDtypeStruct((2, HO), jnp.float32),
    )
    out_specs = (
        pl.BlockSpec((bm, FEAT), row_block),
        pl.BlockSpec((bm, H2), row_block),
        pl.BlockSpec((bm, D2), row_block),
        pl.BlockSpec((bm, HO), row_block),
        pl.BlockSpec((2, HO), const_block),
    )
    in_specs = [
        pl.BlockSpec((bm, F0), row_block),
        pl.BlockSpec((bm, F1), row_block),
        pl.BlockSpec((bm, F2), row_block),
        pl.BlockSpec((bm, F3), row_block),
        pl.BlockSpec((FEAT, H1), const_block),
        pl.BlockSpec((H1,), vec_block),
        pl.BlockSpec((H1, H2), const_block),
        pl.BlockSpec((H2,), vec_block),
        pl.BlockSpec((H2, D1), const_block),
        pl.BlockSpec((D1,), vec_block),
        pl.BlockSpec((D1, D2), const_block),
        pl.BlockSpec((D2,), vec_block),
        pl.BlockSpec((H2, HO), const_block),
        pl.BlockSpec((HO,), vec_block),
    ]
    return pl.pallas_call(
        _mlp_body,
        grid=grid,
        in_specs=in_specs,
        out_specs=out_specs,
        out_shape=out_shapes,
        scratch_shapes=[pltpu.VMEM((2, HO), jnp.float32)],
        interpret=_INTERPRET,
    )(d_vecs, p_embeddings, ecfps_g, gos_g,
      We1, be1, We2, be2, Wdec1, bdec1, Wdec2, bdec2, Wo1, bo1)


def _head_body(h_ref, stats_ref, gamma_ref, beta_ref, Wo2_ref, bo2_ref,
               y_ref, *, inv_b):
    mean = stats_ref[0, :] * inv_b
    var = stats_ref[1, :] * inv_b - mean * mean
    hn = (h_ref[...] - mean) * jax.lax.rsqrt(var + 1e-5) * gamma_ref[...] \
        + beta_ref[...]
    hn = _leaky(hn)
    y_ref[...] = jnp.dot(hn, Wo2_ref[...],
                         preferred_element_type=jnp.float32) + bo2_ref[...]


def _head(h, stats, gamma, beta, Wo2, bo2, bm):
    Bn, HO = h.shape
    grid = (Bn // bm,)
    return pl.pallas_call(
        functools.partial(_head_body, inv_b=1.0 / Bn),
        grid=grid,
        in_specs=[
            pl.BlockSpec((bm, HO), lambda i: (i, 0)),
            pl.BlockSpec((2, HO), lambda i: (0, 0)),
            pl.BlockSpec((HO,), lambda i: (0,)),
            pl.BlockSpec((HO,), lambda i: (0,)),
            pl.BlockSpec((HO, 1), lambda i: (0, 0)),
            pl.BlockSpec((1,), lambda i: (0,)),
        ],
        out_specs=pl.BlockSpec((bm, 1), lambda i: (i, 0)),
        out_shape=jax.ShapeDtypeStruct((Bn, 1), jnp.float32),
        interpret=_INTERPRET,
    )(h, stats, gamma, beta, Wo2, bo2)


_NPAD = 10240


def _newton_rsqrt(x):
    y = jnp.full_like(x, 2.0 ** -10)
    for k in range(9, 0, -1):
        y = jnp.where(x < 4.0 ** k, 2.0 ** -k, y)
    for _ in range(7):
        y = y * (1.5 - 0.5 * x * y * y)
    return y


def _edges3(edge_index, edge_weight, nr):
    e = edge_weight.shape[0]
    epad = nr * 2048
    padi = jnp.zeros((epad - e,), jnp.int32)
    padf = jnp.zeros((epad - e,), jnp.float32)
    src2 = jnp.concatenate([edge_index[0].astype(jnp.int32), padi]).reshape(16, nr, 128)
    dst2 = jnp.concatenate([edge_index[1].astype(jnp.int32), padi]).reshape(16, nr, 128)
    ew2 = jnp.concatenate([edge_weight, padf]).reshape(16, nr, 128)
    return src2, dst2, ew2


def _make_sc_deg(nr_d, nr_p):
    mesh = plsc.VectorSubcoreMesh(core_axis_name="c", subcore_axis_name="s",
                                  num_cores=2, num_subcores=16)

    def body(dstd_hbm, ewd_hbm, dstp_hbm, ewp_hbm, dinvd_hbm, dinvp_hbm,
             dstv, ewv, zv, degl, deg_sh):
        c = lax.axis_index("c")
        s = lax.axis_index("s")

        zero16f = jnp.zeros((16,), jnp.float32)

        def zv_body(i, _):
            zv[pl.ds(i * 16, 16)] = zero16f
            return 0
        lax.fori_loop(0, 128, zv_body, 0)

        for dst_hbm, ew_hbm, nr, dinv_hbm, my_c in (
                (dstd_hbm, ewd_hbm, nr_d, dinvd_hbm, 0),
                (dstp_hbm, ewp_hbm, nr_p, dinvp_hbm, 1)):
            pltpu.sync_copy(dst_hbm.at[s], dstv.at[pl.ds(0, nr)])
            pltpu.sync_copy(ew_hbm.at[s], ewv.at[pl.ds(0, nr)])

            @pl.when(s == 0)
            def _zero_deg():
                for i in range(_NPAD // 2048):
                    pltpu.sync_copy(zv, deg_sh.at[pl.ds(i * 2048, 2048)])

            plsc.subcore_barrier()

            def deg_body(j, _):
                pltpu.sync_copy(ewv.at[j], deg_sh.at[dstv.at[j]], add=True)
                return 0
            lax.fori_loop(0, nr, deg_body, 0)

            plsc.subcore_barrier()

            pltpu.sync_copy(deg_sh.at[pl.ds(s * 640, 640)], degl)

            def dinv_body(i, _):
                x = degl[pl.ds(i * 16, 16)] + 1.0
                degl[pl.ds(i * 16, 16)] = _newton_rsqrt(x)
                return 0
            lax.fori_loop(0, 40, dinv_body, 0)

            @pl.when(c == my_c)
            def _emit():
                pltpu.sync_copy(degl, dinv_hbm.at[pl.ds(s * 640, 640)])

            plsc.subcore_barrier()

    return pl.kernel(
        body,
        out_type=(jax.ShapeDtypeStruct((_NPAD,), jnp.float32),
                  jax.ShapeDtypeStruct((_NPAD,), jnp.float32)),
        mesh=mesh,
        scratch_types=[
            pltpu.VMEM((max(nr_d, nr_p), 128), jnp.int32),
            pltpu.VMEM((max(nr_d, nr_p), 128), jnp.float32),
            pltpu.VMEM((2048,), jnp.float32),
            pltpu.VMEM((640,), jnp.float32),
            pltpu.VMEM_SHARED((_NPAD,), jnp.float32),
        ],
    )


def _make_sc_agg(nr):
    ng = nr * 8
    mesh = plsc.VectorSubcoreMesh(core_axis_name="c", subcore_axis_name="s",
                                  num_cores=2, num_subcores=16)

    def body(hflat_hbm, src_hbm, dst_hbm, ew_hbm, bias_hbm, dinv_hbm,
             out_hbm,
             srcv, dstv, ewv, rows, zrows, facc, fh, dix, biasv, acc_sh):
        c = lax.axis_index("c")
        s = lax.axis_index("s")

        pltpu.sync_copy(src_hbm.at[s], srcv)
        pltpu.sync_copy(dst_hbm.at[s], dstv)
        pltpu.sync_copy(ew_hbm.at[s], ewv)
        pltpu.sync_copy(bias_hbm, biasv)

        zero16f = jnp.zeros((16,), jnp.float32)
        for r in range(16):
            for k in range(8):
                zrows[r, pl.ds(k * 16, 16)] = zero16f

        for step in range(4):
            fc = 2 * step + c
            hbase = fc * _NPAD

            def zt(t, _):
                pltpu.sync_copy(zrows, acc_sh.at[pl.ds(s * 640 + t * 16, 16)])
                return 0
            lax.fori_loop(0, 40, zt, 0)

            plsc.subcore_barrier()

            def sbody(i, _):
                r = i >> 3
                col = (i & 7) * 16
                sv = srcv[r, pl.ds(col, 16)]
                dv = dstv[r, pl.ds(col, 16)]
                wv = ewv[r, pl.ds(col, 16)]
                pltpu.sync_copy(hflat_hbm.at[sv + hbase], rows)
                for r16 in range(16):
                    co = wv[r16]

                    def kb(k, _, r16=r16, co=co):
                        rows[r16, pl.ds(k * 16, 16)] = \
                            rows[r16, pl.ds(k * 16, 16)] * co
                        return 0
                    lax.fori_loop(0, 8, kb, 0)
                pltpu.sync_copy(rows, acc_sh.at[dv], add=True)
                return 0
            lax.fori_loop(0, ng, sbody, 0)

            plsc.subcore_barrier()

            def fbody(gg, _):
                gidx = s + 16 * gg

                @pl.when(gidx < 1250)
                def _():
                    g0 = gidx * 8
                    pltpu.sync_copy(acc_sh.at[pl.ds(g0, 8)], facc)
                    pltpu.sync_copy(hflat_hbm.at[pl.ds(hbase + g0, 8)], fh)
                    pltpu.sync_copy(dinv_hbm.at[pl.ds(g0, 16)], dix)
                    div = dix[pl.ds(0, 16)]
                    for r in range(8):
                        di = div[r]

                        def fk(k, _, r=r, di=di):
                            o = di * (facc[r, pl.ds(k * 16, 16)]
                                      + fh[r, pl.ds(k * 16, 16)]) \
                                + biasv[pl.ds(fc * 128 + k * 16, 16)]
                            facc[r, pl.ds(k * 16, 16)] = \
                                jnp.where(o >= 0, o, 0.01 * o)
                            return 0
                        lax.fori_loop(0, 8, fk, 0)
                    pltpu.sync_copy(
                        facc,
                        out_hbm.at[pl.ds(g0, 8), pl.ds(fc * 128, 128)])
                return 0
            lax.fori_loop(0, 79, fbody, 0)

            plsc.subcore_barrier()

    return pl.kernel(
        body,
        out_type=jax.ShapeDtypeStruct((_NPAD, _F), jnp.float32),
        mesh=mesh,
        scratch_types=[
            pltpu.VMEM((nr, 128), jnp.int32),
            pltpu.VMEM((nr, 128), jnp.int32),
            pltpu.VMEM((nr, 128), jnp.float32),
            pltpu.VMEM((16, 128), jnp.float32),
            pltpu.VMEM((16, 128), jnp.float32),
            pltpu.VMEM((8, 128), jnp.float32),
            pltpu.VMEM((8, 128), jnp.float32),
            pltpu.VMEM((16,), jnp.float32),
            pltpu.VMEM((_F,), jnp.float32),
            pltpu.VMEM_SHARED((_NPAD, 128), jnp.float32),
        ],
    )


def _make_sc_gather(bn):
    rows_per = bn // 32
    grp = 32
    ngrp = rows_per // grp
    mesh = plsc.VectorSubcoreMesh(core_axis_name="c", subcore_axis_name="s", num_cores=2, num_subcores=16)

    def body(tabd_hbm, tabp_hbm, di_hbm, pi_hbm, od_hbm, op_hbm,
             idxv, grows):
        c = lax.axis_index("c")
        s = lax.axis_index("s")
        base = (s * 2 + c) * rows_per
        for idx_hbm, tab_hbm, o_hbm in ((di_hbm, tabd_hbm, od_hbm),
                                        (pi_hbm, tabp_hbm, op_hbm)):
            pltpu.sync_copy(idx_hbm.at[pl.ds(base, rows_per)], idxv)

            def gbody(g, _):
                pltpu.sync_copy(tab_hbm.at[idxv.at[pl.ds(g * grp, grp)]],
                                grows)
                pltpu.sync_copy(grows, o_hbm.at[pl.ds(base + g * grp, grp)])
                return 0
            lax.fori_loop(0, ngrp, gbody, 0)

    return pl.kernel(
        body,
        out_type=(jax.ShapeDtypeStruct((bn, _F), jnp.float32),
                  jax.ShapeDtypeStruct((bn, _F), jnp.float32)),
        mesh=mesh,
        scratch_types=[
            pltpu.VMEM((rows_per,), jnp.int32),
            pltpu.VMEM((grp, _F), jnp.float32),
        ],
    )


def kernel(d_index, p_index, d_vecs, p_embeddings, y,
           d_ecfps, d_edge_index, d_edge_weight,
           p_gos, p_edge_index, p_edge_weight,
           Wd, bd, Wp, bp,
           We1, be1, We2, be2,
           Wdec1, bdec1, Wdec2, bdec2,
           Wo1, bo1, gamma, beta, Wo2, bo2):
    sd, dd, wd = _edges3(d_edge_index, d_edge_weight, 39)
    sp, dp, wp = _edges3(p_edge_index, p_edge_weight, 14)
    dinv_d, dinv_p = _make_sc_deg(39, 14)(dd, wd, dp, wp)
    h_d = _matmul_scaled(d_ecfps, Wd, dinv_d)
    h_p = _matmul_scaled(p_gos, Wp, dinv_p)
    out_d = _make_sc_agg(39)(h_d, sd, dd, wd, bd, dinv_d)
    out_p = _make_sc_agg(14)(h_p, sp, dp, wp, bp, dinv_p)
    ecfps_g, gos_g = _make_sc_gather(d_index.shape[0])(
        out_d, out_p, d_index.astype(jnp.int32), p_index.astype(jnp.int32))

    feature, encoded, decoded, h, stats = _mlp(
        d_vecs, p_embeddings, ecfps_g, gos_g,
        We1, be1, We2, be2, Wdec1, bdec1, Wdec2, bdec2, Wo1, bo1,
        bm=min(256, d_vecs.shape[0]))
    y_out = _head(h, stats, gamma, beta, Wo2, bo2,
                  bm=min(512, d_vecs.shape[0]))
    return (y_out, encoded, decoded, feature)


# speedup vs baseline: 1.6688x; 1.4314x over previous
import functools

import jax
import jax.numpy as jnp
from jax import lax
from jax.experimental import pallas as pl
from jax.experimental.pallas import tpu as pltpu
from jax.experimental.pallas import tpu_sc as plsc

_INTERPRET = False

_N = 10000
_F = 1024
_CH = 2000
_NSTEP = 3


def _leaky(x):
    return jnp.where(x >= 0, x, 0.01 * x)


def _bdot(a, b):
    return jnp.dot(a.astype(jnp.bfloat16), b.astype(jnp.bfloat16),
                   preferred_element_type=jnp.float32)


def _mm_body(x_ref, w_ref, dinv_ref, o_ref):
    f = pl.program_id(1)
    res = jnp.dot(x_ref[...], w_ref[:, pl.ds(f * 128, 128)],
                  preferred_element_type=jnp.float32)
    o_ref[...] = res * dinv_ref[...][:, None]


def _matmul_scaled(x, W, dinv):
    M, K = x.shape
    bm = 512
    nb = pl.cdiv(_NPAD, bm)
    return pl.pallas_call(
        _mm_body,
        grid=(nb, 8),
        in_specs=[pl.BlockSpec((bm, K), lambda i, f: (i, 0)),
                  pl.BlockSpec((K, _F), lambda i, f: (0, 0)),
                  pl.BlockSpec((bm,), lambda i, f: (i,))],
        out_specs=pl.BlockSpec((bm, 128), lambda i, f: (f * nb + i, 0)),
        out_shape=jax.ShapeDtypeStruct((8 * _NPAD, 128), jnp.float32),
        interpret=_INTERPRET,
    )(x.astype(jnp.bfloat16), W.astype(jnp.bfloat16), dinv)


def _mlp_body(dv_ref, pe_ref, ec_ref, go_ref,
              We1_ref, be1_ref, We2_ref, be2_ref,
              Wd1_ref, bd1_ref, Wd2_ref, bd2_ref,
              Wo1_ref, bo1_ref,
              feat_ref, enc_ref, dec_ref, h_ref, stats_ref,
              acc_ref):
    feat = jnp.concatenate(
        [dv_ref[...], pe_ref[...], ec_ref[...], go_ref[...]], axis=1)
    feat_ref[...] = feat
    e1 = _leaky(_bdot(feat, We1_ref[...]) + be1_ref[...])
    enc = _leaky(_bdot(e1, We2_ref[...]) + be2_ref[...])
    enc_ref[...] = enc
    d1 = _leaky(_bdot(enc, Wd1_ref[...]) + bd1_ref[...])
    dec_ref[...] = _leaky(_bdot(d1, Wd2_ref[...]) + bd2_ref[...])
    h = jnp.dot(enc, Wo1_ref[...],
                preferred_element_type=jnp.float32) + bo1_ref[...]
    h_ref[...] = h

    i = pl.program_id(0)

    @pl.when(i == 0)
    def _init():
        acc_ref[...] = jnp.zeros_like(acc_ref)

    acc_ref[0, :] += jnp.sum(h, axis=0)
    acc_ref[1, :] += jnp.sum(h * h, axis=0)

    @pl.when(i == pl.num_programs(0) - 1)
    def _emit():
        stats_ref[...] = acc_ref[...]


def _mlp(d_vecs, p_embeddings, ecfps_g, gos_g,
         We1, be1, We2, be2, Wdec1, bdec1, Wdec2, bdec2, Wo1, bo1, bm):
    Bn = d_vecs.shape[0]
    F0 = d_vecs.shape[1]
    F1 = p_embeddings.shape[1]
    F2 = ecfps_g.shape[1]
    F3 = gos_g.shape[1]
    FEAT = F0 + F1 + F2 + F3
    H1 = We1.shape[1]
    H2 = We2.shape[1]
    D1 = Wdec1.shape[1]
    D2 = Wdec2.shape[1]
    HO = Wo1.shape[1]
    grid = (Bn // bm,)

    def row_block(i):
        return (i, 0)

    def const_block(i):
        return (0, 0)

    def vec_block(i):
        return (0,)

    out_shapes = (
        jax.ShapeDtypeStruct((Bn, FEAT), jnp.float32),
        jax.ShapeDtypeStruct((Bn, H2), jnp.float32),
        jax.ShapeDtypeStruct((Bn, D2), jnp.float32),
        jax.ShapeDtypeStruct((Bn, HO), jnp.float32),
        jax.ShapeDtypeStruct((2, HO), jnp.float32),
    )
    out_specs = (
        pl.BlockSpec((bm, FEAT), row_block),
        pl.BlockSpec((bm, H2), row_block),
        pl.BlockSpec((bm, D2), row_block),
        pl.BlockSpec((bm, HO), row_block),
        pl.BlockSpec((2, HO), const_block),
    )
    in_specs = [
        pl.BlockSpec((bm, F0), row_block),
        pl.BlockSpec((bm, F1), row_block),
        pl.BlockSpec((bm, F2), row_block),
        pl.BlockSpec((bm, F3), row_block),
        pl.BlockSpec((FEAT, H1), const_block),
        pl.BlockSpec((H1,), vec_block),
        pl.BlockSpec((H1, H2), const_block),
        pl.BlockSpec((H2,), vec_block),
        pl.BlockSpec((H2, D1), const_block),
        pl.BlockSpec((D1,), vec_block),
        pl.BlockSpec((D1, D2), const_block),
        pl.BlockSpec((D2,), vec_block),
        pl.BlockSpec((H2, HO), const_block),
        pl.BlockSpec((HO,), vec_block),
    ]
    return pl.pallas_call(
        _mlp_body,
        grid=grid,
        in_specs=in_specs,
        out_specs=out_specs,
        out_shape=out_shapes,
        scratch_shapes=[pltpu.VMEM((2, HO), jnp.float32)],
        interpret=_INTERPRET,
    )(d_vecs, p_embeddings, ecfps_g, gos_g,
      We1, be1, We2, be2, Wdec1, bdec1, Wdec2, bdec2, Wo1, bo1)


def _head_body(h_ref, stats_ref, gamma_ref, beta_ref, Wo2_ref, bo2_ref,
               y_ref, *, inv_b):
    mean = stats_ref[0, :] * inv_b
    var = stats_ref[1, :] * inv_b - mean * mean
    hn = (h_ref[...] - mean) * jax.lax.rsqrt(var + 1e-5) * gamma_ref[...] \
        + beta_ref[...]
    hn = _leaky(hn)
    y_ref[...] = jnp.dot(hn, Wo2_ref[...],
                         preferred_element_type=jnp.float32) + bo2_ref[...]


def _head(h, stats, gamma, beta, Wo2, bo2, bm):
    Bn, HO = h.shape
    grid = (Bn // bm,)
    return pl.pallas_call(
        functools.partial(_head_body, inv_b=1.0 / Bn),
        grid=grid,
        in_specs=[
            pl.BlockSpec((bm, HO), lambda i: (i, 0)),
            pl.BlockSpec((2, HO), lambda i: (0, 0)),
            pl.BlockSpec((HO,), lambda i: (0,)),
            pl.BlockSpec((HO,), lambda i: (0,)),
            pl.BlockSpec((HO, 1), lambda i: (0, 0)),
            pl.BlockSpec((1,), lambda i: (0,)),
        ],
        out_specs=pl.BlockSpec((bm, 1), lambda i: (i, 0)),
        out_shape=jax.ShapeDtypeStruct((Bn, 1), jnp.float32),
        interpret=_INTERPRET,
    )(h, stats, gamma, beta, Wo2, bo2)


_NPAD = 10240


def _newton_rsqrt(x):
    y = jnp.full_like(x, 2.0 ** -10)
    for k in range(9, 0, -1):
        y = jnp.where(x < 4.0 ** k, 2.0 ** -k, y)
    for _ in range(7):
        y = y * (1.5 - 0.5 * x * y * y)
    return y


def _edges3(edge_index, edge_weight, nr4):
    e = edge_weight.shape[0] + _N
    epad = nr4 * 256
    loop = jnp.arange(_N, dtype=jnp.int32)
    padi = jnp.zeros((epad - e,), jnp.int32)
    padf = jnp.zeros((epad - e,), jnp.float32)
    ones = jnp.ones((_N,), jnp.float32)
    src2 = jnp.concatenate([edge_index[0].astype(jnp.int32), loop, padi]).reshape(16, nr4 // 8, 128)
    dst2 = jnp.concatenate([edge_index[1].astype(jnp.int32), loop, padi]).reshape(16, nr4 // 8, 128)
    ew2 = jnp.concatenate([edge_weight, ones, padf]).reshape(16, nr4 // 8, 128)
    return src2, dst2, ew2


def _make_sc_deg(nr_d, nr_p):
    mesh = plsc.VectorSubcoreMesh(core_axis_name="c", subcore_axis_name="s",
                                  num_cores=2, num_subcores=16)

    def body(dstd_hbm, ewd_hbm, dstp_hbm, ewp_hbm, dinvd_hbm, dinvp_hbm,
             dstv, ewv, zv, degl, deg_sh):
        c = lax.axis_index("c")
        s = lax.axis_index("s")

        zero16f = jnp.zeros((16,), jnp.float32)

        def zv_body(i, _):
            zv[pl.ds(i * 16, 16)] = zero16f
            return 0
        lax.fori_loop(0, 128, zv_body, 0)

        for dst_hbm, ew_hbm, nr, dinv_hbm, my_c in (
                (dstd_hbm, ewd_hbm, nr_d, dinvd_hbm, 0),
                (dstp_hbm, ewp_hbm, nr_p, dinvp_hbm, 1)):
            nr16 = nr // 8
            pltpu.sync_copy(dst_hbm.at[s], dstv.at[pl.ds(0, nr16)])
            pltpu.sync_copy(ew_hbm.at[s], ewv.at[pl.ds(0, nr16)])

            @pl.when(s == 0)
            def _zero_deg():
                for i in range(_NPAD // 2048):
                    pltpu.sync_copy(zv, deg_sh.at[pl.ds(i * 2048, 2048)])

            plsc.subcore_barrier()

            def deg_body(j, _):
                pltpu.sync_copy(ewv.at[j], deg_sh.at[dstv.at[j]], add=True)
                return 0
            lax.fori_loop(0, nr16, deg_body, 0)

            plsc.subcore_barrier()

            pltpu.sync_copy(deg_sh.at[pl.ds(s * 640, 640)], degl)

            def dinv_body(i, _):
                x = degl[pl.ds(i * 16, 16)]
                degl[pl.ds(i * 16, 16)] = _newton_rsqrt(jnp.maximum(x, 1.0))
                return 0
            lax.fori_loop(0, 40, dinv_body, 0)

            @pl.when(c == my_c)
            def _emit():
                pltpu.sync_copy(degl, dinv_hbm.at[pl.ds(s * 640, 640)])

            plsc.subcore_barrier()

    return pl.kernel(
        body,
        out_type=(jax.ShapeDtypeStruct((_NPAD,), jnp.float32),
                  jax.ShapeDtypeStruct((_NPAD,), jnp.float32)),
        mesh=mesh,
        scratch_types=[
            pltpu.VMEM((max(nr_d, nr_p) // 8, 128), jnp.int32),
            pltpu.VMEM((max(nr_d, nr_p) // 8, 128), jnp.float32),
            pltpu.VMEM((2048,), jnp.float32),
            pltpu.VMEM((640,), jnp.float32),
            pltpu.VMEM_SHARED((_NPAD,), jnp.float32),
        ],
    )


def _make_sc_agg(nr4):
    nr16 = nr4 // 8
    npair = nr4 // 2
    mesh = plsc.VectorSubcoreMesh(core_axis_name="c", subcore_axis_name="s",
                                  num_cores=2, num_subcores=16)

    def body(hflat_hbm, src_hbm, dst_hbm, ew_hbm, bias_hbm, dinv_hbm,
             out_hbm,
             srcv, dstv, ewv, rowsA, rowsB, facc, dix,
             bias128, acc_sh, gsA, gsB, ssA, ssB):
        c = lax.axis_index("c")
        s = lax.axis_index("s")

        pltpu.sync_copy(src_hbm.at[s], srcv)
        pltpu.sync_copy(dst_hbm.at[s], dstv)
        pltpu.sync_copy(ew_hbm.at[s], ewv)

        zero16f = jnp.zeros((16,), jnp.float32)

        def rb0(j, _):
            def rk(k, _, j=j):
                srcv[j, pl.ds(k * 16, 16)] = \
                    srcv[j, pl.ds(k * 16, 16)] + c * _NPAD
                return 0
            lax.fori_loop(0, 8, rk, 0)
            return 0
        lax.fori_loop(0, nr16, rb0, 0)

        def scale32(rows, j):
            for q in range(1):
                ewq = ewv[j >> 3, pl.ds((j & 7) * 16, 16)]
                for r16 in range(16):
                    co = ewq[r16]
                    rr = r16

                    def kb(k, _, rr=rr, co=co):
                        rows[rr, pl.ds(k * 16, 16)] = \
                            rows[rr, pl.ds(k * 16, 16)] * co
                        return 0
                    lax.fori_loop(0, 8, kb, 0)

        def chunk_body(step, _):
            fc = 2 * step + c
            hbase = fc * _NPAD
            pltpu.sync_copy(bias_hbm.at[pl.ds(fc * 128, 128)], bias128)

            for r in range(8):
                for k in range(8):
                    facc[r, pl.ds(k * 16, 16)] = zero16f

            def zt(t, _):
                pltpu.sync_copy(facc, acc_sh.at[pl.ds(s * 632 + t * 8, 8)])
                return 0
            lax.fori_loop(0, 79, zt, 0)

            plsc.subcore_barrier()

            def sidx(g):
                return srcv[g >> 3, pl.ds((g & 7) * 16, 16)]

            def didx(g):
                return dstv[g >> 3, pl.ds((g & 7) * 16, 16)]

            pltpu.async_copy(hflat_hbm.at[sidx(0)], rowsA, gsA)
            pltpu.async_copy(hflat_hbm.at[sidx(1)], rowsB, gsB)

            def waitdma(buf, sem):
                pltpu.make_async_copy(hflat_hbm.at[pl.ds(0, 16)], buf,
                                      sem).wait()

            def pbody(jj, _):
                j0 = 2 * jj
                waitdma(rowsA, gsA)
                scale32(rowsA, j0)
                pltpu.async_copy(rowsA, acc_sh.at[didx(j0)], ssA,
                                 add=True)
                waitdma(rowsB, gsB)
                scale32(rowsB, j0 + 1)
                pltpu.async_copy(rowsB, acc_sh.at[didx(j0 + 1)], ssB,
                                 add=True)

                @pl.when(jj < npair - 1)
                def _more():
                    waitdma(rowsA, ssA)
                    pltpu.async_copy(hflat_hbm.at[sidx(j0 + 2)], rowsA,
                                     gsA)
                    waitdma(rowsB, ssB)
                    pltpu.async_copy(hflat_hbm.at[sidx(j0 + 3)], rowsB,
                                     gsB)

                @pl.when(jj == npair - 1)
                def _last():
                    waitdma(rowsA, ssA)
                    waitdma(rowsB, ssB)
                return 0
            lax.fori_loop(0, npair, pbody, 0)

            plsc.subcore_barrier()

            def fbody(gg, _):
                gidx = s + 16 * gg
                g0 = gidx * 8
                pltpu.sync_copy(acc_sh.at[pl.ds(g0, 8)], facc)
                pltpu.sync_copy(dinv_hbm.at[pl.ds(g0, 16)], dix)
                div = dix[pl.ds(0, 16)]
                if True:
                    for r in range(8):
                        di = div[r]

                        def fk(k, _, r=r, di=di):
                            o = di * facc[r, pl.ds(k * 16, 16)] \
                                + bias128[pl.ds(k * 16, 16)]
                            facc[r, pl.ds(k * 16, 16)] = \
                                jnp.where(o >= 0, o, 0.01 * o)
                            return 0
                        lax.fori_loop(0, 8, fk, 0)
                    pltpu.sync_copy(facc, out_hbm.at[pl.ds(hbase + g0, 8)])
                return 0
            lax.fori_loop(0, 79, fbody, 0)

            plsc.subcore_barrier()

            def rbn(j, _):
                def rk(k, _, j=j):
                    srcv[j, pl.ds(k * 16, 16)] = \
                        srcv[j, pl.ds(k * 16, 16)] + 2 * _NPAD
                    return 0
                lax.fori_loop(0, 8, rk, 0)
                return 0
            lax.fori_loop(0, nr16, rbn, 0)
            return 0

        lax.fori_loop(0, 4, chunk_body, 0)

    return pl.kernel(
        body,
        out_type=jax.ShapeDtypeStruct((8 * _NPAD, 128), jnp.float32),
        mesh=mesh,
        scratch_types=[
            pltpu.VMEM((nr16, 128), jnp.int32),
            pltpu.VMEM((nr16, 128), jnp.int32),
            pltpu.VMEM((nr16, 128), jnp.float32),
            pltpu.VMEM((16, 128), jnp.float32),
            pltpu.VMEM((16, 128), jnp.float32),
            pltpu.VMEM((8, 128), jnp.float32),
            pltpu.VMEM((16,), jnp.float32),
            pltpu.VMEM((128,), jnp.float32),
            pltpu.VMEM_SHARED((10112, 128), jnp.float32),
            pltpu.SemaphoreType.DMA,
            pltpu.SemaphoreType.DMA,
            pltpu.SemaphoreType.DMA,
            pltpu.SemaphoreType.DMA,
        ],
    )


def _reasm_body(a_ref, o_ref):
    o_ref[...] = a_ref[...]


def _reassemble(cm):
    bm = 512
    nb = _NPAD // bm
    return pl.pallas_call(
        _reasm_body,
        grid=(nb, 8),
        in_specs=[pl.BlockSpec((bm, 128), lambda i, f: (f * nb + i, 0))],
        out_specs=pl.BlockSpec((bm, 128), lambda i, f: (i, f)),
        out_shape=jax.ShapeDtypeStruct((_NPAD, _F), jnp.float32),
        interpret=_INTERPRET,
    )(cm)


def _make_sc_gather(bn):
    rows_per = bn // 32
    grp = 32
    ngrp = rows_per // grp
    mesh = plsc.VectorSubcoreMesh(core_axis_name="c", subcore_axis_name="s", num_cores=2, num_subcores=16)

    def body(tabd_hbm, tabp_hbm, di_hbm, pi_hbm, od_hbm, op_hbm,
             idxv, grows):
        c = lax.axis_index("c")
        s = lax.axis_index("s")
        base = (s * 2 + c) * rows_per
        for idx_hbm, tab_hbm, o_hbm in ((di_hbm, tabd_hbm, od_hbm),
                                        (pi_hbm, tabp_hbm, op_hbm)):
            pltpu.sync_copy(idx_hbm.at[pl.ds(base, rows_per)], idxv)

            def gbody(g, _):
                pltpu.sync_copy(tab_hbm.at[idxv.at[pl.ds(g * grp, grp)]],
                                grows)
                pltpu.sync_copy(grows, o_hbm.at[pl.ds(base + g * grp, grp)])
                return 0
            lax.fori_loop(0, ngrp, gbody, 0)

    return pl.kernel(
        body,
        out_type=(jax.ShapeDtypeStruct((bn, _F), jnp.float32),
                  jax.ShapeDtypeStruct((bn, _F), jnp.float32)),
        mesh=mesh,
        scratch_types=[
            pltpu.VMEM((rows_per,), jnp.int32),
            pltpu.VMEM((grp, _F), jnp.float32),
        ],
    )


def kernel(d_index, p_index, d_vecs, p_embeddings, y,
           d_ecfps, d_edge_index, d_edge_weight,
           p_gos, p_edge_index, p_edge_weight,
           Wd, bd, Wp, bp,
           We1, be1, We2, be2,
           Wdec1, bdec1, Wdec2, bdec2,
           Wo1, bo1, gamma, beta, Wo2, bo2):
    sd, dd, wd = _edges3(d_edge_index, d_edge_weight, 344)
    sp, dp, wp = _edges3(p_edge_index, p_edge_weight, 152)
    dinv_d, dinv_p = _make_sc_deg(344, 152)(dd, wd, dp, wp)
    h_d = _matmul_scaled(d_ecfps, Wd, dinv_d)
    h_p = _matmul_scaled(p_gos, Wp, dinv_p)
    out_d = _reassemble(_make_sc_agg(344)(h_d, sd, dd, wd, bd, dinv_d))
    out_p = _reassemble(_make_sc_agg(152)(h_p, sp, dp, wp, bp, dinv_p))
    ecfps_g, gos_g = _make_sc_gather(d_index.shape[0])(
        out_d, out_p, d_index.astype(jnp.int32), p_index.astype(jnp.int32))

    feature, encoded, decoded, h, stats = _mlp(
        d_vecs, p_embeddings, ecfps_g, gos_g,
        We1, be1, We2, be2, Wdec1, bdec1, Wdec2, bdec2, Wo1, bo1,
        bm=min(256, d_vecs.shape[0]))
    y_out = _head(h, stats, gamma, beta, Wo2, bo2,
                  bm=min(512, d_vecs.shape[0]))
    return (y_out, encoded, decoded, feature)


# speedup vs baseline: 1.8701x; 1.1207x over previous
import functools

import jax
import jax.numpy as jnp
from jax import lax
from jax.experimental import pallas as pl
from jax.experimental.pallas import tpu as pltpu
from jax.experimental.pallas import tpu_sc as plsc

_INTERPRET = False

_N = 10000
_F = 1024
_CH = 2000
_NSTEP = 3


def _leaky(x):
    return jnp.where(x >= 0, x, 0.01 * x)


def _bdot(a, b):
    return jnp.dot(a.astype(jnp.bfloat16), b.astype(jnp.bfloat16),
                   preferred_element_type=jnp.float32)


def _mm_body(x_ref, w_ref, dinv_ref, o_ref):
    f = pl.program_id(1)
    res = jnp.dot(x_ref[...], w_ref[:, pl.ds(f * 128, 128)],
                  preferred_element_type=jnp.float32)
    o_ref[...] = res * dinv_ref[...][:, None]


def _matmul_scaled(x, W, dinv):
    M, K = x.shape
    bm = 512
    nb = pl.cdiv(_NPAD, bm)
    return pl.pallas_call(
        _mm_body,
        grid=(nb, 8),
        in_specs=[pl.BlockSpec((bm, K), lambda i, f: (i, 0)),
                  pl.BlockSpec((K, _F), lambda i, f: (0, 0)),
                  pl.BlockSpec((bm,), lambda i, f: (i,))],
        out_specs=pl.BlockSpec((bm, 128), lambda i, f: (f * nb + i, 0)),
        out_shape=jax.ShapeDtypeStruct((8 * _NPAD, 128), jnp.float32),
        interpret=_INTERPRET,
    )(x.astype(jnp.bfloat16), W.astype(jnp.bfloat16), dinv)


def _mlp_body(dv_ref, pe_ref, ec_ref, go_ref,
              We1_ref, be1_ref, We2_ref, be2_ref,
              Wd1_ref, bd1_ref, Wd2_ref, bd2_ref,
              Wo1_ref, bo1_ref,
              feat_ref, enc_ref, dec_ref, h_ref, stats_ref,
              acc_ref):
    feat = jnp.concatenate(
        [dv_ref[...], pe_ref[...], ec_ref[...], go_ref[...]], axis=1)
    feat_ref[...] = feat
    e1 = _leaky(_bdot(feat, We1_ref[...]) + be1_ref[...])
    enc = _leaky(_bdot(e1, We2_ref[...]) + be2_ref[...])
    enc_ref[...] = enc
    d1 = _leaky(_bdot(enc, Wd1_ref[...]) + bd1_ref[...])
    dec_ref[...] = _leaky(_bdot(d1, Wd2_ref[...]) + bd2_ref[...])
    h = jnp.dot(enc, Wo1_ref[...],
                preferred_element_type=jnp.float32) + bo1_ref[...]
    h_ref[...] = h

    i = pl.program_id(0)

    @pl.when(i == 0)
    def _init():
        acc_ref[...] = jnp.zeros_like(acc_ref)

    acc_ref[0, :] += jnp.sum(h, axis=0)
    acc_ref[1, :] += jnp.sum(h * h, axis=0)

    @pl.when(i == pl.num_programs(0) - 1)
    def _emit():
        stats_ref[...] = acc_ref[...]


def _mlp(d_vecs, p_embeddings, ecfps_g, gos_g,
         We1, be1, We2, be2, Wdec1, bdec1, Wdec2, bdec2, Wo1, bo1, bm):
    Bn = d_vecs.shape[0]
    F0 = d_vecs.shape[1]
    F1 = p_embeddings.shape[1]
    F2 = ecfps_g.shape[1]
    F3 = gos_g.shape[1]
    FEAT = F0 + F1 + F2 + F3
    H1 = We1.shape[1]
    H2 = We2.shape[1]
    D1 = Wdec1.shape[1]
    D2 = Wdec2.shape[1]
    HO = Wo1.shape[1]
    grid = (Bn // bm,)

    def row_block(i):
        return (i, 0)

    def const_block(i):
        return (0, 0)

    def vec_block(i):
        return (0,)

    out_shapes = (
        jax.ShapeDtypeStruct((Bn, FEAT), jnp.float32),
        jax.ShapeDtypeStruct((Bn, H2), jnp.float32),
        jax.ShapeDtypeStruct((Bn, D2), jnp.float32),
        jax.ShapeDtypeStruct((Bn, HO), jnp.float32),
        jax.ShapeDtypeStruct((2, HO), jnp.float32),
    )
    out_specs = (
        pl.BlockSpec((bm, FEAT), row_block),
        pl.BlockSpec((bm, H2), row_block),
        pl.BlockSpec((bm, D2), row_block),
        pl.BlockSpec((bm, HO), row_block),
        pl.BlockSpec((2, HO), const_block),
    )
    in_specs = [
        pl.BlockSpec((bm, F0), row_block),
        pl.BlockSpec((bm, F1), row_block),
        pl.BlockSpec((bm, F2), row_block),
        pl.BlockSpec((bm, F3), row_block),
        pl.BlockSpec((FEAT, H1), const_block),
        pl.BlockSpec((H1,), vec_block),
        pl.BlockSpec((H1, H2), const_block),
        pl.BlockSpec((H2,), vec_block),
        pl.BlockSpec((H2, D1), const_block),
        pl.BlockSpec((D1,), vec_block),
        pl.BlockSpec((D1, D2), const_block),
        pl.BlockSpec((D2,), vec_block),
        pl.BlockSpec((H2, HO), const_block),
        pl.BlockSpec((HO,), vec_block),
    ]
    return pl.pallas_call(
        _mlp_body,
        grid=grid,
        in_specs=in_specs,
        out_specs=out_specs,
        out_shape=out_shapes,
        scratch_shapes=[pltpu.VMEM((2, HO), jnp.float32)],
        interpret=_INTERPRET,
    )(d_vecs, p_embeddings, ecfps_g, gos_g,
      We1, be1, We2, be2, Wdec1, bdec1, Wdec2, bdec2, Wo1, bo1)


def _head_body(h_ref, stats_ref, gamma_ref, beta_ref, Wo2_ref, bo2_ref,
               y_ref, *, inv_b):
    mean = stats_ref[0, :] * inv_b
    var = stats_ref[1, :] * inv_b - mean * mean
    hn = (h_ref[...] - mean) * jax.lax.rsqrt(var + 1e-5) * gamma_ref[...] \
        + beta_ref[...]
    hn = _leaky(hn)
    y_ref[...] = jnp.dot(hn, Wo2_ref[...],
                         preferred_element_type=jnp.float32) + bo2_ref[...]


def _head(h, stats, gamma, beta, Wo2, bo2, bm):
    Bn, HO = h.shape
    grid = (Bn // bm,)
    return pl.pallas_call(
        functools.partial(_head_body, inv_b=1.0 / Bn),
        grid=grid,
        in_specs=[
            pl.BlockSpec((bm, HO), lambda i: (i, 0)),
            pl.BlockSpec((2, HO), lambda i: (0, 0)),
            pl.BlockSpec((HO,), lambda i: (0,)),
            pl.BlockSpec((HO,), lambda i: (0,)),
            pl.BlockSpec((HO, 1), lambda i: (0, 0)),
            pl.BlockSpec((1,), lambda i: (0,)),
        ],
        out_specs=pl.BlockSpec((bm, 1), lambda i: (i, 0)),
        out_shape=jax.ShapeDtypeStruct((Bn, 1), jnp.float32),
        interpret=_INTERPRET,
    )(h, stats, gamma, beta, Wo2, bo2)


_NPAD = 10240


def _newton_rsqrt(x):
    y = jnp.full_like(x, 2.0 ** -10)
    for k in range(9, 0, -1):
        y = jnp.where(x < 4.0 ** k, 2.0 ** -k, y)
    for _ in range(7):
        y = y * (1.5 - 0.5 * x * y * y)
    return y


def _edges3(edge_index, edge_weight, nr4):
    e = edge_weight.shape[0] + _N
    epad = nr4 * 256
    loop = jnp.arange(_N, dtype=jnp.int32)
    padi = jnp.zeros((epad - e,), jnp.int32)
    padf = jnp.zeros((epad - e,), jnp.float32)
    ones = jnp.ones((_N,), jnp.float32)
    src2 = jnp.concatenate([edge_index[0].astype(jnp.int32), loop, padi]).reshape(16, nr4 // 8, 128)
    dst2 = jnp.concatenate([edge_index[1].astype(jnp.int32), loop, padi]).reshape(16, nr4 // 8, 128)
    ew2 = jnp.concatenate([edge_weight, ones, padf]).reshape(16, nr4 // 8, 128)
    return src2, dst2, ew2


def _make_sc_deg(nr_d, nr_p):
    mesh = plsc.VectorSubcoreMesh(core_axis_name="c", subcore_axis_name="s",
                                  num_cores=2, num_subcores=16)

    def body(dstd_hbm, ewd_hbm, dstp_hbm, ewp_hbm, dinvd_hbm, dinvp_hbm,
             dstv, ewv, zv, degl, deg_sh):
        c = lax.axis_index("c")
        s = lax.axis_index("s")

        zero16f = jnp.zeros((16,), jnp.float32)

        def zv_body(i, _):
            zv[pl.ds(i * 16, 16)] = zero16f
            return 0
        lax.fori_loop(0, 128, zv_body, 0)

        for dst_hbm, ew_hbm, nr, dinv_hbm, my_c in (
                (dstd_hbm, ewd_hbm, nr_d, dinvd_hbm, 0),
                (dstp_hbm, ewp_hbm, nr_p, dinvp_hbm, 1)):
            nr16 = nr // 8
            pltpu.sync_copy(dst_hbm.at[s], dstv.at[pl.ds(0, nr16)])
            pltpu.sync_copy(ew_hbm.at[s], ewv.at[pl.ds(0, nr16)])

            @pl.when(s == 0)
            def _zero_deg():
                for i in range(_NPAD // 2048):
                    pltpu.sync_copy(zv, deg_sh.at[pl.ds(i * 2048, 2048)])

            plsc.subcore_barrier()

            def deg_body(j, _):
                pltpu.sync_copy(ewv.at[j], deg_sh.at[dstv.at[j]], add=True)
                return 0
            lax.fori_loop(0, nr16, deg_body, 0)

            plsc.subcore_barrier()

            pltpu.sync_copy(deg_sh.at[pl.ds(s * 640, 640)], degl)

            def dinv_body(i, _):
                x = degl[pl.ds(i * 16, 16)]
                degl[pl.ds(i * 16, 16)] = _newton_rsqrt(jnp.maximum(x, 1.0))
                return 0
            lax.fori_loop(0, 40, dinv_body, 0)

            @pl.when(c == my_c)
            def _emit():
                pltpu.sync_copy(degl, dinv_hbm.at[pl.ds(s * 640, 640)])

            plsc.subcore_barrier()

    return pl.kernel(
        body,
        out_type=(jax.ShapeDtypeStruct((_NPAD,), jnp.float32),
                  jax.ShapeDtypeStruct((_NPAD,), jnp.float32)),
        mesh=mesh,
        scratch_types=[
            pltpu.VMEM((max(nr_d, nr_p) // 8, 128), jnp.int32),
            pltpu.VMEM((max(nr_d, nr_p) // 8, 128), jnp.float32),
            pltpu.VMEM((2048,), jnp.float32),
            pltpu.VMEM((640,), jnp.float32),
            pltpu.VMEM_SHARED((_NPAD,), jnp.float32),
        ],
    )


def _make_sc_agg(nr4):
    nr16 = nr4 // 8
    npair = nr4 // 2
    mesh = plsc.VectorSubcoreMesh(core_axis_name="c", subcore_axis_name="s",
                                  num_cores=2, num_subcores=16)

    def body(hflat_hbm, src_hbm, dst_hbm, ew_hbm, bias_hbm, dinv_hbm,
             out_hbm,
             srcv, dstv, ewv, rowsA, rowsB, facc, dinvv,
             bias128, acc_sh, gsA, gsB, ssA, ssB):
        c = lax.axis_index("c")
        s = lax.axis_index("s")

        pltpu.sync_copy(src_hbm.at[s], srcv)
        pltpu.sync_copy(dst_hbm.at[s], dstv)
        pltpu.sync_copy(ew_hbm.at[s], ewv)
        pltpu.sync_copy(dinv_hbm, dinvv)

        zero16f = jnp.zeros((16,), jnp.float32)

        def rb0(j, _):
            def rk(k, _, j=j):
                srcv[j, pl.ds(k * 16, 16)] = \
                    srcv[j, pl.ds(k * 16, 16)] + c * _NPAD
                return 0
            lax.fori_loop(0, 8, rk, 0)
            return 0
        lax.fori_loop(0, nr16, rb0, 0)

        def scale32(rows, j):
            for q in range(1):
                ewq = ewv[j >> 3, pl.ds((j & 7) * 16, 16)]
                for r16 in range(16):
                    co = ewq[r16]
                    rr = r16

                    def kb(k, _, rr=rr, co=co):
                        rows[rr, pl.ds(k * 16, 16)] = \
                            rows[rr, pl.ds(k * 16, 16)] * co
                        return 0
                    lax.fori_loop(0, 8, kb, 0)

        def chunk_body(step, _):
            fc = 2 * step + c
            hbase = fc * _NPAD
            pltpu.sync_copy(bias_hbm.at[pl.ds(fc * 128, 128)], bias128)

            for r in range(16):
                for k in range(8):
                    facc[r, pl.ds(k * 16, 16)] = zero16f

            def zt(t, _):
                pltpu.sync_copy(facc,
                                acc_sh.at[pl.ds(s * 632 + t * 16, 16)])
                return 0
            lax.fori_loop(0, 39, zt, 0)
            pltpu.sync_copy(facc.at[pl.ds(0, 8)],
                            acc_sh.at[pl.ds(s * 632 + 624, 8)])

            plsc.subcore_barrier()

            def sidx(g):
                return srcv[g >> 3, pl.ds((g & 7) * 16, 16)]

            def didx(g):
                return dstv[g >> 3, pl.ds((g & 7) * 16, 16)]

            pltpu.async_copy(hflat_hbm.at[sidx(0)], rowsA, gsA)
            pltpu.async_copy(hflat_hbm.at[sidx(1)], rowsB, gsB)

            def waitdma(buf, sem):
                pltpu.make_async_copy(hflat_hbm.at[pl.ds(0, 16)], buf,
                                      sem).wait()

            def pbody(jj, _):
                j0 = 2 * jj
                waitdma(rowsA, gsA)
                scale32(rowsA, j0)
                pltpu.async_copy(rowsA, acc_sh.at[didx(j0)], ssA,
                                 add=True)
                waitdma(rowsB, gsB)
                scale32(rowsB, j0 + 1)
                pltpu.async_copy(rowsB, acc_sh.at[didx(j0 + 1)], ssB,
                                 add=True)

                @pl.when(jj < npair - 1)
                def _more():
                    waitdma(rowsA, ssA)
                    pltpu.async_copy(hflat_hbm.at[sidx(j0 + 2)], rowsA,
                                     gsA)
                    waitdma(rowsB, ssB)
                    pltpu.async_copy(hflat_hbm.at[sidx(j0 + 3)], rowsB,
                                     gsB)

                @pl.when(jj == npair - 1)
                def _last():
                    waitdma(rowsA, ssA)
                    waitdma(rowsB, ssB)
                return 0
            lax.fori_loop(0, npair, pbody, 0)

            plsc.subcore_barrier()

            def fbody(gg, _):
                gidx = s + 16 * gg

                @pl.when(gidx < 632)
                def _():
                    g0 = gidx * 16
                    pltpu.sync_copy(acc_sh.at[pl.ds(g0, 16)], facc)
                    div = dinvv[pl.ds(g0, 16)]
                    for r in range(16):
                        di = div[r]

                        def fk(k, _, r=r, di=di):
                            o = di * facc[r, pl.ds(k * 16, 16)] \
                                + bias128[pl.ds(k * 16, 16)]
                            facc[r, pl.ds(k * 16, 16)] = \
                                jnp.where(o >= 0, o, 0.01 * o)
                            return 0
                        lax.fori_loop(0, 8, fk, 0)
                    pltpu.sync_copy(facc,
                                    out_hbm.at[pl.ds(hbase + g0, 16)])
                return 0
            lax.fori_loop(0, 40, fbody, 0)

            plsc.subcore_barrier()

            def rbn(j, _):
                def rk(k, _, j=j):
                    srcv[j, pl.ds(k * 16, 16)] = \
                        srcv[j, pl.ds(k * 16, 16)] + 2 * _NPAD
                    return 0
                lax.fori_loop(0, 8, rk, 0)
                return 0
            lax.fori_loop(0, nr16, rbn, 0)
            return 0

        lax.fori_loop(0, 4, chunk_body, 0)

    return pl.kernel(
        body,
        out_type=jax.ShapeDtypeStruct((8 * _NPAD, 128), jnp.float32),
        mesh=mesh,
        scratch_types=[
            pltpu.VMEM((nr16, 128), jnp.int32),
            pltpu.VMEM((nr16, 128), jnp.int32),
            pltpu.VMEM((nr16, 128), jnp.float32),
            pltpu.VMEM((16, 128), jnp.float32),
            pltpu.VMEM((16, 128), jnp.float32),
            pltpu.VMEM((16, 128), jnp.float32),
            pltpu.VMEM((10240,), jnp.float32),
            pltpu.VMEM((128,), jnp.float32),
            pltpu.VMEM_SHARED((10112, 128), jnp.float32),
            pltpu.SemaphoreType.DMA,
            pltpu.SemaphoreType.DMA,
            pltpu.SemaphoreType.DMA,
            pltpu.SemaphoreType.DMA,
        ],
    )


def _reasm_body(a_ref, o_ref):
    o_ref[...] = a_ref[...]


def _reassemble(cm):
    bm = 512
    nb = _NPAD // bm
    return pl.pallas_call(
        _reasm_body,
        grid=(nb, 8),
        in_specs=[pl.BlockSpec((bm, 128), lambda i, f: (f * nb + i, 0))],
        out_specs=pl.BlockSpec((bm, 128), lambda i, f: (i, f)),
        out_shape=jax.ShapeDtypeStruct((_NPAD, _F), jnp.float32),
        interpret=_INTERPRET,
    )(cm)


def _make_sc_gather(bn):
    rows_per = bn // 32
    grp = 32
    ngrp = rows_per // grp
    mesh = plsc.VectorSubcoreMesh(core_axis_name="c", subcore_axis_name="s", num_cores=2, num_subcores=16)

    def body(tabd_hbm, tabp_hbm, di_hbm, pi_hbm, od_hbm, op_hbm,
             idxv, grows):
        c = lax.axis_index("c")
        s = lax.axis_index("s")
        base = (s * 2 + c) * rows_per
        for idx_hbm, tab_hbm, o_hbm in ((di_hbm, tabd_hbm, od_hbm),
                                        (pi_hbm, tabp_hbm, op_hbm)):
            pltpu.sync_copy(idx_hbm.at[pl.ds(base, rows_per)], idxv)

            def gbody(g, _):
                pltpu.sync_copy(tab_hbm.at[idxv.at[pl.ds(g * grp, grp)]],
                                grows)
                pltpu.sync_copy(grows, o_hbm.at[pl.ds(base + g * grp, grp)])
                return 0
            lax.fori_loop(0, ngrp, gbody, 0)

    return pl.kernel(
        body,
        out_type=(jax.ShapeDtypeStruct((bn, _F), jnp.float32),
                  jax.ShapeDtypeStruct((bn, _F), jnp.float32)),
        mesh=mesh,
        scratch_types=[
            pltpu.VMEM((rows_per,), jnp.int32),
            pltpu.VMEM((grp, _F), jnp.float32),
        ],
    )


def kernel(d_index, p_index, d_vecs, p_embeddings, y,
           d_ecfps, d_edge_index, d_edge_weight,
           p_gos, p_edge_index, p_edge_weight,
           Wd, bd, Wp, bp,
           We1, be1, We2, be2,
           Wdec1, bdec1, Wdec2, bdec2,
           Wo1, bo1, gamma, beta, Wo2, bo2):
    sd, dd, wd = _edges3(d_edge_index, d_edge_weight, 344)
    sp, dp, wp = _edges3(p_edge_index, p_edge_weight, 152)
    dinv_d, dinv_p = _make_sc_deg(344, 152)(dd, wd, dp, wp)
    h_d = _matmul_scaled(d_ecfps, Wd, dinv_d)
    h_p = _matmul_scaled(p_gos, Wp, dinv_p)
    out_d = _reassemble(_make_sc_agg(344)(h_d, sd, dd, wd, bd, dinv_d))
    out_p = _reassemble(_make_sc_agg(152)(h_p, sp, dp, wp, bp, dinv_p))
    ecfps_g, gos_g = _make_sc_gather(d_index.shape[0])(
        out_d, out_p, d_index.astype(jnp.int32), p_index.astype(jnp.int32))

    feature, encoded, decoded, h, stats = _mlp(
        d_vecs, p_embeddings, ecfps_g, gos_g,
        We1, be1, We2, be2, Wdec1, bdec1, Wdec2, bdec2, Wo1, bo1,
        bm=min(256, d_vecs.shape[0]))
    y_out = _head(h, stats, gamma, beta, Wo2, bo2,
                  bm=min(512, d_vecs.shape[0]))
    return (y_out, encoded, decoded, feature)
